# baseline (speedup 1.0000x reference)
import jax
import jax.numpy as jnp
import numpy as np
from jax.experimental import pallas as pl
from jax.experimental.pallas import tpu as pltpu
from jax.sharding import Mesh, PartitionSpec as P


def _body2(x_ref, a0_ref, a1_ref, w_ref, out_ref, h_ref):
    @pl.when(pl.program_id(0) == 0)
    def _():
        h_ref[...] = jnp.dot(
            x_ref[...], w_ref[...], preferred_element_type=jnp.float32
        )

    bm = a0_ref.shape[0]
    out_ref[:bm, :] = jnp.dot(
        a0_ref[...], h_ref[...], preferred_element_type=jnp.float32
    )
    out_ref[bm:, :] = jnp.dot(
        a1_ref[...], h_ref[...], preferred_element_type=jnp.float32
    )


def _body1(x_ref, a_ref, w_ref, out_ref, h_ref):
    @pl.when(pl.program_id(0) == 0)
    def _():
        h_ref[...] = jnp.dot(
            x_ref[...], w_ref[...], preferred_element_type=jnp.float32
        )

    out_ref[...] = jnp.dot(
        a_ref[...], h_ref[...], preferred_element_type=jnp.float32
    )


def _pick_bm(m, streams):
    best = 0
    for cand in range(8, min(m, 2048) + 1, 8):
        if m % (streams * cand) == 0 and \
                cand * 10000 * 4 * 2 * streams <= 40 * 1024 * 1024:
            best = cand
    return best


def _local(x, A_loc, W):
    m = A_loc.shape[0]
    N, d_in = x.shape
    d_out = W.shape[1]

    bm2 = _pick_bm(m, 2)
    bm1 = _pick_bm(m, 1)
    h_scratch = pltpu.VMEM((N, d_out), jnp.float32)

    if bm2 >= bm1:
        bm = bm2
        return pl.pallas_call(
            _body2,
            grid=(m // (2 * bm),),
            in_specs=[
                pl.BlockSpec((N, d_in), lambda i: (0, 0)),
                pl.BlockSpec((bm, N), lambda i: (2 * i, 0)),
                pl.BlockSpec((bm, N), lambda i: (2 * i + 1, 0)),
                pl.BlockSpec((d_in, d_out), lambda i: (0, 0)),
            ],
            out_specs=pl.BlockSpec((2 * bm, d_out), lambda i: (i, 0)),
            out_shape=jax.ShapeDtypeStruct((m, d_out), jnp.float32),
            scratch_shapes=[h_scratch],
        )(x, A_loc, A_loc, W)

    bm = bm1
    return pl.pallas_call(
        _body1,
        grid=(m // bm,),
        in_specs=[
            pl.BlockSpec((N, d_in), lambda i: (0, 0)),
            pl.BlockSpec((bm, N), lambda i: (i, 0)),
            pl.BlockSpec((d_in, d_out), lambda i: (0, 0)),
        ],
        out_specs=pl.BlockSpec((bm, d_out), lambda i: (i, 0)),
        out_shape=jax.ShapeDtypeStruct((m, d_out), jnp.float32),
        scratch_shapes=[h_scratch],
    )(x, A_loc, W)


def kernel(x, A, W):
    N = A.shape[0]
    devs = jax.devices()
    n_dev = len(devs)
    if n_dev > 1 and N % n_dev == 0 and _pick_bm(N // n_dev, 1) >= 8:
        mesh = Mesh(np.array(devs), ("r",))
        f = jax.shard_map(
            _local,
            mesh=mesh,
            in_specs=(P(None, None), P("r", None), P(None, None)),
            out_specs=P("r", None),
            check_vma=False,
        )
        return f(x, A, W)
    return _local(x, A, W)


# speedup vs baseline: 5.9581x; 5.9581x over previous
import jax
import jax.numpy as jnp
from jax.experimental import pallas as pl
from jax.experimental.pallas import tpu as pltpu


def _body2(x_ref, a0_ref, a1_ref, w_ref, out_ref, h_ref):
    @pl.when(pl.program_id(0) == 0)
    def _():
        h_ref[...] = jnp.dot(
            x_ref[...], w_ref[...], preferred_element_type=jnp.float32
        )

    bm = a0_ref.shape[0]
    out_ref[:bm, :] = jnp.dot(
        a0_ref[...], h_ref[...], preferred_element_type=jnp.float32
    )
    out_ref[bm:, :] = jnp.dot(
        a1_ref[...], h_ref[...], preferred_element_type=jnp.float32
    )


def _body1(x_ref, a_ref, w_ref, out_ref, h_ref):
    @pl.when(pl.program_id(0) == 0)
    def _():
        h_ref[...] = jnp.dot(
            x_ref[...], w_ref[...], preferred_element_type=jnp.float32
        )

    out_ref[...] = jnp.dot(
        a_ref[...], h_ref[...], preferred_element_type=jnp.float32
    )


def _pick_bm(m, streams):
    best = 0
    for cand in range(8, min(m, 2048) + 1, 8):
        if m % (streams * cand) == 0 and \
                cand * 10000 * 4 * 2 * streams <= 40 * 1024 * 1024:
            best = cand
    return best


def _local(x, A_loc, W):
    m = A_loc.shape[0]
    N, d_in = x.shape
    d_out = W.shape[1]

    bm2 = _pick_bm(m, 2)
    bm1 = _pick_bm(m, 1)
    h_scratch = pltpu.VMEM((N, d_out), jnp.float32)

    if bm2 >= bm1:
        bm = bm2
        return pl.pallas_call(
            _body2,
            grid=(m // (2 * bm),),
            in_specs=[
                pl.BlockSpec((N, d_in), lambda i: (0, 0)),
                pl.BlockSpec((bm, N), lambda i: (2 * i, 0)),
                pl.BlockSpec((bm, N), lambda i: (2 * i + 1, 0)),
                pl.BlockSpec((d_in, d_out), lambda i: (0, 0)),
            ],
            out_specs=pl.BlockSpec((2 * bm, d_out), lambda i: (i, 0)),
            out_shape=jax.ShapeDtypeStruct((m, d_out), jnp.float32),
            scratch_shapes=[h_scratch],
        )(x, A_loc, A_loc, W)

    bm = bm1
    return pl.pallas_call(
        _body1,
        grid=(m // bm,),
        in_specs=[
            pl.BlockSpec((N, d_in), lambda i: (0, 0)),
            pl.BlockSpec((bm, N), lambda i: (i, 0)),
            pl.BlockSpec((d_in, d_out), lambda i: (0, 0)),
        ],
        out_specs=pl.BlockSpec((bm, d_out), lambda i: (i, 0)),
        out_shape=jax.ShapeDtypeStruct((m, d_out), jnp.float32),
        scratch_shapes=[h_scratch],
    )(x, A_loc, W)


def kernel(x, A, W):
    return _local(x, A, W)
